# baseline (speedup 1.0000x reference)
import functools

import jax
import jax.numpy as jnp
from jax import lax
from jax.experimental import pallas as pl
from jax.experimental.pallas import tpu as pltpu
from jax.experimental.pallas import tpu_sc as plsc

NC = 2
NS = 16
L = 16
NW = NC * NS

R = 32768
D = 1024
HALF = R // 2
RPW = R // NW
CH = 32
NCHUNK = RPW // CH

_mesh = plsc.VectorSubcoreMesh(core_axis_name="c", subcore_axis_name="s")


@functools.partial(
    pl.kernel,
    mesh=_mesh,
    out_type=jax.ShapeDtypeStruct((R, D), jnp.float32),
    scratch_types=[
        pltpu.VMEM((RPW,), jnp.float32),
        pltpu.VMEM((L,), jnp.float32),
        pltpu.VMEM((NCHUNK, CH), jnp.int32),
        pltpu.VMEM((CH, D), jnp.float32),
        pltpu.SemaphoreType.DMA,
    ],
)
def _exchange(x_hbm, mask_hbm, thr_hbm, out_hbm, mask_v, thr_v, idx_v, buf, sem):
    wid = lax.axis_index("s") * NC + lax.axis_index("c")
    base = wid * RPW
    is1 = jnp.int32(wid >= NW // 2)
    off_true = is1 * HALF
    off_false = HALF - off_true
    rbase = base - is1 * HALF

    pltpu.sync_copy(mask_hbm.at[pl.ds(base, RPW)], mask_v)
    pltpu.sync_copy(thr_hbm, thr_v)
    thr = thr_v[...]

    iota = lax.iota(jnp.int32, L)
    for j in range(RPW // L):
        m = mask_v[pl.ds(j * L, L)]
        sel = jnp.where(m >= thr, off_true, off_false)
        idx_v[j // (CH // L), pl.ds((j % (CH // L)) * L, L)] = (
            rbase + j * L + iota + sel
        )

    def body(i, carry):
        pltpu.async_copy(x_hbm.at[idx_v.at[i]], buf, sem).wait()
        obase = pl.multiple_of(base + i * CH, 8)
        pltpu.sync_copy(buf, out_hbm.at[pl.ds(obase, CH)])
        return carry

    lax.fori_loop(0, NCHUNK, body, 0)


def kernel(x, mask, mask_threshold):
    xf = x.reshape(R, D)
    mf = mask.reshape(R)
    thr = jnp.full((L,), mask_threshold, dtype=jnp.float32)
    out = _exchange(xf, mf, thr)
    x0 = out[:HALF].reshape(2, 8192, D)
    x1 = out[HALF:].reshape(2, 8192, D)
    return (x0, x1)


# speedup vs baseline: 1.1100x; 1.1100x over previous
import functools

import jax
import jax.numpy as jnp
from jax import lax
from jax.experimental import pallas as pl
from jax.experimental.pallas import tpu as pltpu
from jax.experimental.pallas import tpu_sc as plsc

NC = 2
NS = 16
L = 16
NW = NC * NS

R = 32768
D = 1024
HALF = R // 2
RPW = R // NW
CH = 32
NCHUNK = RPW // CH

_mesh = plsc.VectorSubcoreMesh(core_axis_name="c", subcore_axis_name="s")


NBUF = 2
G = NCHUNK // NBUF


@functools.partial(
    pl.kernel,
    mesh=_mesh,
    out_type=jax.ShapeDtypeStruct((R, D), jnp.float32),
    scratch_types=[
        pltpu.VMEM((RPW,), jnp.float32),
        pltpu.VMEM((L,), jnp.float32),
        pltpu.VMEM((NCHUNK, CH), jnp.int32),
        pltpu.VMEM((CH, D), jnp.float32),
        pltpu.VMEM((CH, D), jnp.float32),
        pltpu.SemaphoreType.DMA,
        pltpu.SemaphoreType.DMA,
        pltpu.SemaphoreType.DMA,
        pltpu.SemaphoreType.DMA,
    ],
)
def _exchange(x_hbm, mask_hbm, thr_hbm, out_hbm, mask_v, thr_v, idx_v,
              buf0, buf1, gsem0, gsem1, ssem0, ssem1):
    wid = lax.axis_index("s") * NC + lax.axis_index("c")
    base = wid * RPW
    is1 = jnp.int32(wid >= NW // 2)
    off_true = is1 * HALF
    off_false = HALF - off_true
    rbase = base - is1 * HALF

    pltpu.sync_copy(mask_hbm.at[pl.ds(base, RPW)], mask_v)
    pltpu.sync_copy(thr_hbm, thr_v)
    thr = thr_v[...]

    iota = lax.iota(jnp.int32, L)
    for j in range(RPW // L):
        m = mask_v[pl.ds(j * L, L)]
        sel = jnp.where(m >= thr, off_true, off_false)
        idx_v[j // (CH // L), pl.ds((j % (CH // L)) * L, L)] = (
            rbase + j * L + iota + sel
        )

    bufs = (buf0, buf1)
    gsems = (gsem0, gsem1)
    ssems = (ssem0, ssem1)

    def out_at(i):
        return out_hbm.at[pl.ds(pl.multiple_of(base + i * CH, 8), CH)]

    def start_gather(i, b):
        pltpu.async_copy(x_hbm.at[idx_v.at[i]], bufs[b], gsems[b])

    def wait_gather(i, b):
        pltpu.make_async_copy(x_hbm.at[idx_v.at[i]], bufs[b], gsems[b]).wait()

    def start_store(i, b):
        pltpu.async_copy(bufs[b], out_at(i), ssems[b])

    def wait_store(i, b):
        pltpu.make_async_copy(bufs[b], out_at(i), ssems[b]).wait()

    for b in range(NBUF):
        start_gather(b, b)

    def body(g, carry):
        for b in range(NBUF):
            i = g * NBUF + b
            wait_gather(i, b)
            start_store(i, b)
            wait_store(i, b)
            start_gather(i + NBUF, b)
        return carry

    lax.fori_loop(0, G - 1, body, 0)

    for b in range(NBUF):
        i = (G - 1) * NBUF + b
        wait_gather(i, b)
        start_store(i, b)
    for b in range(NBUF):
        wait_store((G - 1) * NBUF + b, b)


def kernel(x, mask, mask_threshold):
    xf = x.reshape(R, D)
    mf = mask.reshape(R)
    thr = jnp.full((L,), mask_threshold, dtype=jnp.float32)
    out = _exchange(xf, mf, thr)
    x0 = out[:HALF].reshape(2, 8192, D)
    x1 = out[HALF:].reshape(2, 8192, D)
    return (x0, x1)


# speedup vs baseline: 1.9187x; 1.7285x over previous
import functools

import jax
import jax.numpy as jnp
from jax import lax
from jax.experimental import pallas as pl
from jax.experimental.pallas import tpu as pltpu
from jax.experimental.pallas import tpu_sc as plsc

NC = 2
NS = 16
L = 16
NW = NC * NS

R = 32768
D = 1024
HALF = R // 2
SPAN = HALF // NW
CH = 32
NCH = SPAN // CH

_mesh = plsc.VectorSubcoreMesh(core_axis_name="c", subcore_axis_name="s")


@functools.partial(
    pl.kernel,
    mesh=_mesh,
    out_type=(
        jax.ShapeDtypeStruct((HALF, D), jnp.float32),
        jax.ShapeDtypeStruct((HALF, D), jnp.float32),
    ),
    scratch_types=[
        pltpu.VMEM((2 * SPAN,), jnp.float32),
        pltpu.VMEM((L,), jnp.float32),
        pltpu.VMEM((2 * NCH, CH), jnp.int32),
        pltpu.VMEM((CH, D), jnp.float32),
        pltpu.VMEM((CH, D), jnp.float32),
        pltpu.SemaphoreType.DMA,
        pltpu.SemaphoreType.DMA,
        pltpu.SemaphoreType.DMA,
        pltpu.SemaphoreType.DMA,
    ],
)
def _exchange_sc(x_hbm, mask_hbm, thr_hbm, out0_hbm, out1_hbm, mask_v, thr_v,
                 idx_v, buf0, buf1, gsem0, gsem1, ssem0, ssem1):
    wid = lax.axis_index("s") * NC + lax.axis_index("c")
    base = wid * SPAN

    pltpu.sync_copy(mask_hbm.at[pl.ds(base, SPAN)], mask_v.at[pl.ds(0, SPAN)])
    pltpu.sync_copy(mask_hbm.at[pl.ds(HALF + base, SPAN)],
                    mask_v.at[pl.ds(SPAN, SPAN)])
    pltpu.sync_copy(thr_hbm, thr_v)
    thr = thr_v[...]

    iota = lax.iota(jnp.int32, L)
    for p in range(2):
        off_t = HALF if p else 0
        off_f = 0 if p else HALF
        for j in range(SPAN // L):
            m = mask_v[pl.ds(p * SPAN + j * L, L)]
            src = base + j * L + iota + jnp.where(m >= thr, off_t, off_f)
            idx_v[p * NCH + j // (CH // L), pl.ds((j % (CH // L)) * L, L)] = src

    bufs = (buf0, buf1)
    gsems = (gsem0, gsem1)
    ssems = (ssem0, ssem1)
    outs = (out0_hbm, out1_hbm)

    def out_at(p, c):
        return outs[p].at[pl.ds(pl.multiple_of(base + c * CH, 8), CH)]

    def idx_at(p, c):
        return idx_v.at[p * NCH + c]

    def start_gather(p, c):
        pltpu.async_copy(x_hbm.at[idx_at(p, c)], bufs[p], gsems[p])

    def wait_gather(p, c):
        pltpu.make_async_copy(x_hbm.at[idx_at(p, c)], bufs[p],
                              gsems[p]).wait()

    def start_store(p, c):
        pltpu.async_copy(bufs[p], out_at(p, c), ssems[p])

    def wait_store(p, c):
        pltpu.make_async_copy(bufs[p], out_at(p, c), ssems[p]).wait()

    for p in range(2):
        start_gather(p, 0)

    def body(g, carry):
        for p in range(2):
            wait_gather(p, g)
            start_store(p, g)
            wait_store(p, g)
            start_gather(p, g + 1)
        return carry

    lax.fori_loop(0, NCH - 1, body, 0)

    for p in range(2):
        wait_gather(p, NCH - 1)
        start_store(p, NCH - 1)
    for p in range(2):
        wait_store(p, NCH - 1)


def kernel(x, mask, mask_threshold):
    xf = x.reshape(R, D)
    mf = mask.reshape(R)
    thr = jnp.full((L,), mask_threshold, dtype=jnp.float32)
    o0, o1 = _exchange_sc(xf, mf, thr)
    return (o0.reshape(2, 8192, D), o1.reshape(2, 8192, D))


# speedup vs baseline: 1.9270x; 1.0043x over previous
import functools

import jax
import jax.numpy as jnp
from jax import lax
from jax.experimental import pallas as pl
from jax.experimental.pallas import tpu as pltpu
from jax.experimental.pallas import tpu_sc as plsc

NC = 2
NS = 16
L = 16
NW = NC * NS

R = 32768
D = 1024
HALF = R // 2
SPAN = HALF // NW
CH = 32
NCH = SPAN // CH

_mesh = plsc.VectorSubcoreMesh(core_axis_name="c", subcore_axis_name="s")


@functools.partial(
    pl.kernel,
    mesh=_mesh,
    out_type=(
        jax.ShapeDtypeStruct((HALF, D), jnp.float32),
        jax.ShapeDtypeStruct((HALF, D), jnp.float32),
    ),
    scratch_types=[
        pltpu.VMEM((2 * SPAN,), jnp.float32),
        pltpu.VMEM((L,), jnp.float32),
        pltpu.VMEM((2 * NCH, CH), jnp.int32),
        pltpu.VMEM((CH, D), jnp.float32),
        pltpu.VMEM((CH, D), jnp.float32),
        pltpu.SemaphoreType.DMA,
        pltpu.SemaphoreType.DMA,
        pltpu.SemaphoreType.DMA,
        pltpu.SemaphoreType.DMA,
    ],
)
def _exchange_sc(x_hbm, mask_hbm, thr_hbm, out0_hbm, out1_hbm, mask_v, thr_v,
                 idx_v, buf0, buf1, gsem0, gsem1, ssem0, ssem1):
    wid = lax.axis_index("s") * NC + lax.axis_index("c")
    base = wid * SPAN

    pltpu.sync_copy(mask_hbm.at[pl.ds(base, SPAN)], mask_v.at[pl.ds(0, SPAN)])
    pltpu.sync_copy(mask_hbm.at[pl.ds(HALF + base, SPAN)],
                    mask_v.at[pl.ds(SPAN, SPAN)])
    pltpu.sync_copy(thr_hbm, thr_v)
    thr = thr_v[...]

    iota = lax.iota(jnp.int32, L)
    for p in range(2):
        off_t = HALF if p else 0
        off_f = 0 if p else HALF
        for j in range(SPAN // L):
            m = mask_v[pl.ds(p * SPAN + j * L, L)]
            src = base + j * L + iota + jnp.where(m >= thr, off_t, off_f)
            idx_v[p * NCH + j // (CH // L), pl.ds((j % (CH // L)) * L, L)] = src

    bufs = (buf0, buf1)
    gsems = (gsem0, gsem1)
    ssems = (ssem0, ssem1)
    NBUF = 2
    G = NCH // NBUF

    for p, out_hbm in enumerate((out0_hbm, out1_hbm)):

        def out_at(c):
            return out_hbm.at[pl.ds(pl.multiple_of(base + c * CH, 8), CH)]

        def idx_at(c):
            return idx_v.at[p * NCH + c]

        def start_gather(c, b):
            pltpu.async_copy(x_hbm.at[idx_at(c)], bufs[b], gsems[b])

        def wait_gather(c, b):
            pltpu.make_async_copy(x_hbm.at[idx_at(c)], bufs[b],
                                  gsems[b]).wait()

        def start_store(c, b):
            pltpu.async_copy(bufs[b], out_at(c), ssems[b])

        def wait_store(c, b):
            pltpu.make_async_copy(bufs[b], out_at(c), ssems[b]).wait()

        for b in range(NBUF):
            start_gather(b, b)

        def body(g, carry):
            for b in range(NBUF):
                c = g * NBUF + b
                wait_gather(c, b)
                start_store(c, b)
                wait_store(c, b)
                start_gather(c + NBUF, b)
            return carry

        lax.fori_loop(0, G - 1, body, 0)

        for b in range(NBUF):
            c = (G - 1) * NBUF + b
            wait_gather(c, b)
            start_store(c, b)
        for b in range(NBUF):
            wait_store((G - 1) * NBUF + b, b)


def kernel(x, mask, mask_threshold):
    xf = x.reshape(R, D)
    mf = mask.reshape(R)
    thr = jnp.full((L,), mask_threshold, dtype=jnp.float32)
    o0, o1 = _exchange_sc(xf, mf, thr)
    return (o0.reshape(2, 8192, D), o1.reshape(2, 8192, D))
